# Initial kernel scaffold; baseline (speedup 1.0000x reference)
import jax
import jax.numpy as jnp
from jax.experimental import pallas as pl


def kernel(x, edge_index, batch, W1, b1, W2, b2, W3, b3, W4, b4, Wl, bl):
    raise NotImplementedError("write your pallas kernel here")


# speedup vs baseline: 23.1587x; 23.1587x over previous
import functools

import jax
import jax.numpy as jnp
from jax import lax
from jax.experimental import pallas as pl
from jax.experimental.pallas import tpu as pltpu
from jax.experimental.pallas import tpu_sc as plsc

N = 10000
NP = 10240
E = 320000
CHUNK = 128
EPAD = 2560 * CHUNK
NCH = EPAD // CHUNK
TCH = NCH // 16
TRASH = 10016
NG = 16
NBUF = 4

_mesh = plsc.VectorSubcoreMesh(core_axis_name="c", subcore_axis_name="s")

_f32 = jnp.float32
_i32 = jnp.int32


def _sc_body(rows_hbm, cols_hbm, batch_hbm,
             w0o, w1o, w2o, w3o, w4o, dego, go,
             rows_t, cols_t, deg_p, g_p, dinv_t, wbuf, zbuf, mbuf,
             bvmem, dinv_own, dinv2_own, ident,
             w_acc, deg_sh, dinv_sh, g_sh,
             sem0, sem1, sem2, sem3):
    c = lax.axis_index("c")
    s = lax.axis_index("s")
    sems = (sem0, sem1, sem2, sem3)
    iota = lax.iota(_i32, 16)
    ones = jnp.ones((16,), _f32)
    zeros16 = jnp.zeros((16,), _f32)

    pltpu.sync_copy(rows_hbm.at[pl.ds(s * TCH, TCH)], rows_t)
    pltpu.sync_copy(cols_hbm.at[pl.ds(s * TCH, TCH)], cols_t)
    pltpu.sync_copy(batch_hbm.at[pl.ds(s * 640, 640)], bvmem)

    def _init(o, _):
        for j in range(16):
            zbuf[o * 16 + j, :] = zeros16
        return 0
    lax.fori_loop(0, 40, _init, 0)

    def _init2(o, _):
        deg_p[pl.ds(o * 16, 16)] = zeros16
        g_p[pl.ds(o * 16, 16)] = zeros16
        return 0
    lax.fori_loop(0, 640, _init2, 0)

    for cc in range(8):
        for j in range(5):
            ident[cc, pl.ds(j * 16, 16)] = iota + (cc * 80 + j * 16)

    pltpu.sync_copy(zbuf.at[pl.ds(0, 40)], deg_sh.at[pl.ds(s * 40, 40)])
    pltpu.sync_copy(zbuf.at[pl.ds(0, 40)], g_sh.at[pl.ds(s * 40, 40)])
    pltpu.sync_copy(zbuf, w_acc.at[pl.ds(s * 640, 640)])
    plsc.subcore_barrier()

    def _deg_body(ch, _):
        for j in range(8):
            rv = rows_t[ch, pl.ds(j * 16, 16)]
            plsc.addupdate_scatter(deg_p, [rv], ones)
        return 0
    lax.fori_loop(0, TCH, _deg_body, 0)

    def _stage_deg(r, _):
        wbuf[r, :] = deg_p[pl.ds(r * 16, 16)]
        return 0
    lax.fori_loop(0, 640, _stage_deg, 0)
    for cc in range(8):
        pltpu.sync_copy(wbuf.at[pl.ds(cc * 80, 80)],
                        deg_sh.at[ident.at[cc]], add=True)
    plsc.subcore_barrier()

    pltpu.sync_copy(deg_sh.at[pl.ds(s * 40, 40)], wbuf.at[pl.ds(0, 40)])

    def _dinv_body(r, _):
        x = wbuf[r, :]
        xs = jnp.where(x > 0.0, x, 1.0)
        i = plsc.bitcast(xs, _i32)
        y = plsc.bitcast(0x5F3759DF - lax.shift_right_logical(i, 1), _f32)
        for _ in range(3):
            y = y * (1.5 - 0.5 * xs * y * y)
        dv = jnp.where(x > 0.0, y, 0.0)
        mbuf[0, r, :] = dv
        dinv_own[pl.ds(r * 16, 16)] = dv
        dinv2_own[pl.ds(r * 16, 16)] = dv * dv
        return 0
    lax.fori_loop(0, 40, _dinv_body, 0)

    pltpu.sync_copy(wbuf.at[pl.ds(0, 40)], dego.at[c].at[pl.ds(s * 40, 40)])
    pltpu.sync_copy(mbuf.at[0].at[pl.ds(0, 40)],
                    dinv_sh.at[pl.ds(s * 40, 40)])
    plsc.subcore_barrier()
    pltpu.sync_copy(dinv_sh, wbuf)

    def _stage_dinv(r, _):
        dinv_t[pl.ds(r * 16, 16)] = wbuf[r, :]
        return 0
    lax.fori_loop(0, 640, _stage_dinv, 0)

    def _g_body(ch, _):
        for j in range(8):
            rv = rows_t[ch, pl.ds(j * 16, 16)]
            cv = cols_t[ch, pl.ds(j * 16, 16)]
            vals = plsc.load_gather(dinv_t, [rv])
            plsc.addupdate_scatter(g_p, [cv], vals)
        return 0
    lax.fori_loop(0, TCH, _g_body, 0)

    def _stage_g(r, _):
        wbuf[r, :] = g_p[pl.ds(r * 16, 16)]
        return 0
    lax.fori_loop(0, 640, _stage_g, 0)
    for cc in range(8):
        pltpu.sync_copy(wbuf.at[pl.ds(cc * 80, 80)],
                        g_sh.at[ident.at[cc]], add=True)
    plsc.subcore_barrier()
    pltpu.sync_copy(g_sh.at[pl.ds(s * 40, 40)], mbuf.at[0].at[pl.ds(0, 40)])
    pltpu.sync_copy(mbuf.at[0].at[pl.ds(0, 40)], go.at[c].at[pl.ds(s * 40, 40)])

    def _w0_body(o, _):
        bv = bvmem[pl.ds(o * 16, 16)]
        dvv = dinv_own[pl.ds(o * 16, 16)]
        for j in range(16):
            wbuf[o * 16 + j, :] = jnp.where(iota == bv[j], dvv[j], 0.0)
        return 0
    lax.fori_loop(0, 40, _w0_body, 0)
    pltpu.sync_copy(wbuf, w0o.at[c].at[pl.ds(s * 640, 640)])
    plsc.subcore_barrier()

    passes = [(w0o, w1o), (w1o, w2o), (w2o, w3o), (w3o, w4o)]
    for (wsrc, wdst) in passes:
        pltpu.sync_copy(zbuf, w_acc.at[pl.ds(s * 640, 640)])
        plsc.subcore_barrier()

        def _start(ch, i):
            pltpu.async_copy(wsrc.at[c].at[cols_t.at[ch]], mbuf.at[i],
                             sems[i])

        def _finish(ch, i):
            pltpu.make_async_copy(wsrc.at[c].at[cols_t.at[ch]], mbuf.at[i],
                                  sems[i]).wait()
            pltpu.sync_copy(mbuf.at[i], w_acc.at[rows_t.at[ch]], add=True)

        for i in range(NBUF):
            _start(i, i)

        def _edge_body(o, _):
            for i in range(NBUF):
                ch = o * NBUF + i
                _finish(ch, i)
                _start(ch + NBUF, i)
            return 0
        lax.fori_loop(0, TCH // NBUF - 1, _edge_body, 0)
        for i in range(NBUF):
            _finish(TCH - NBUF + i, i)
        plsc.subcore_barrier()

        pltpu.sync_copy(w_acc.at[pl.ds(s * 640, 640)], wbuf)

        def _scale_body(o, _):
            d2v = dinv2_own[pl.ds(o * 16, 16)]
            for j in range(16):
                i = o * 16 + j
                wbuf[i, :] = wbuf[i, :] * d2v[j]
            return 0
        lax.fori_loop(0, 40, _scale_body, 0)
        pltpu.sync_copy(wbuf, wdst.at[c].at[pl.ds(s * 640, 640)])
        plsc.subcore_barrier()


@jax.jit
def _sc_graph(rows2d, cols2d, batch_pad):
    out_type = (
        jax.ShapeDtypeStruct((2, NP, 16), _f32),
        jax.ShapeDtypeStruct((2, NP, 16), _f32),
        jax.ShapeDtypeStruct((2, NP, 16), _f32),
        jax.ShapeDtypeStruct((2, NP, 16), _f32),
        jax.ShapeDtypeStruct((2, NP, 16), _f32),
        jax.ShapeDtypeStruct((2, 640, 16), _f32),
        jax.ShapeDtypeStruct((2, 640, 16), _f32),
    )
    scratch = [
        pltpu.VMEM((TCH, CHUNK), _i32),
        pltpu.VMEM((TCH, CHUNK), _i32),
        pltpu.VMEM((NP,), _f32),
        pltpu.VMEM((NP,), _f32),
        pltpu.VMEM((NP,), _f32),
        pltpu.VMEM((640, 16), _f32),
        pltpu.VMEM((640, 16), _f32),
        pltpu.VMEM((NBUF, CHUNK, 16), _f32),
        pltpu.VMEM((640,), _i32),
        pltpu.VMEM((640,), _f32),
        pltpu.VMEM((640,), _f32),
        pltpu.VMEM((8, 80), _i32),
        pltpu.VMEM_SHARED((NP, 16), _f32),
        pltpu.VMEM_SHARED((640, 16), _f32),
        pltpu.VMEM_SHARED((640, 16), _f32),
        pltpu.VMEM_SHARED((640, 16), _f32),
        pltpu.SemaphoreType.DMA,
        pltpu.SemaphoreType.DMA,
        pltpu.SemaphoreType.DMA,
        pltpu.SemaphoreType.DMA,
    ]
    return pl.kernel(
        _sc_body, out_type, mesh=_mesh, scratch_types=scratch,
        compiler_params=pltpu.CompilerParams(
            needs_layout_passes=False, use_tc_tiling_on_sc=False),
        name="gcn_sc_passes")(rows2d, cols2d, batch_pad)


def _tc_body(x_ref, b2d_ref, w0_ref, w1_ref, w2_ref, w3_ref, w4_ref,
             deg_ref, g_ref, W1_ref, b1_ref, W2_ref, b2_ref, W3_ref,
             b3_ref, W4_ref, b4_ref, Wl_ref, bl_ref, out_ref):
    def _mt(a, b):
        return lax.dot_general(a, b, (((1,), (1,)), ((), ())),
                               preferred_element_type=_f32)

    def _col(w, v):
        return lax.dot_general(w, v, (((0,), (0,)), ((), ())),
                               preferred_element_type=_f32)

    g = g_ref[...]
    qc0 = _col(w0_ref[...], g)
    qc1 = _col(w1_ref[...], g)
    qc2 = _col(w2_ref[...], g)
    qc3 = _col(w3_ref[...], g)
    w4s = w4_ref[...] * jnp.sqrt(deg_ref[...])
    M = lax.dot_general(w4s, x_ref[...], (((0,), (0,)), ((), ())),
                        preferred_element_type=_f32)
    W1m, W2m, W3m, W4m = W1_ref[...], W2_ref[...], W3_ref[...], W4_ref[...]
    Wc = W4m @ (W3m @ (W2m @ W1m))
    b1r, b2r, b3r, b4r = b1_ref[...], b2_ref[...], b3_ref[...], b4_ref[...]
    beta1 = _mt(_mt(_mt(b1r, W2m), W3m), W4m)
    beta2 = _mt(_mt(b2r, W3m), W4m)
    beta3 = _mt(b3r, W4m)
    pool = (_mt(M, Wc)
            + qc3 * beta1 + qc2 * beta2 + qc1 * beta3
            + qc0 * b4r)
    b2d = b2d_ref[...]
    n_max = jnp.max(jnp.sum(
        (b2d[:, :, None] == lax.broadcasted_iota(_i32, (1, 1, NG), 2))
        .astype(_f32), axis=(0, 1)))
    out_ref[...] = (jnp.sum((pool / n_max) * Wl_ref[...], axis=1,
                            keepdims=True) + bl_ref[...])


@jax.jit
def _tc_assemble(xpad, batch2d, w0, w1, w2, w3, w4, deg, g,
                 W1, b1, W2, b2, W3, b3, W4, b4, Wl, bl):
    return pl.pallas_call(
        _tc_body,
        out_shape=jax.ShapeDtypeStruct((NG, 1), _f32),
    )(xpad, batch2d, w0, w1, w2, w3, w4, deg, g,
      W1, b1, W2, b2, W3, b3, W4, b4, Wl, bl)


def kernel(x, edge_index, batch, W1, b1, W2, b2, W3, b3, W4, b4, Wl, bl):
    row = edge_index[0].astype(_i32)
    col = edge_index[1].astype(_i32)
    pad = jnp.full((EPAD - E,), TRASH, _i32)
    rows2d = jnp.concatenate([row, pad]).reshape(NCH, CHUNK)
    cols2d = jnp.concatenate([col, pad]).reshape(NCH, CHUNK)
    batch_pad = jnp.concatenate(
        [batch.astype(_i32), jnp.full((NP - N,), -1, _i32)])

    w0, w1, w2, w3, w4, dego, go = _sc_graph(rows2d, cols2d, batch_pad)

    xpad = jnp.zeros((NP, 128), _f32).at[:N].set(x)
    out = _tc_assemble(
        xpad, batch_pad.reshape(640, 16),
        w0[0], w1[0], w2[0], w3[0], w4[0],
        dego[0].reshape(NP, 1), go[0].reshape(NP, 1),
        W1, b1.reshape(1, 64), W2, b2.reshape(1, 64),
        W3, b3.reshape(1, 64), W4, b4.reshape(1, 64),
        Wl, bl.reshape(1, 1))
    return out
